# Initial kernel scaffold; baseline (speedup 1.0000x reference)
import jax
import jax.numpy as jnp
from jax.experimental import pallas as pl


def kernel(x, edge_index, W1_l, b1, W1_r, W2_l, b2, W2_r):
    raise NotImplementedError("write your pallas kernel here")


# speedup vs baseline: 10.4537x; 10.4537x over previous
import functools

import jax
import jax.numpy as jnp
from jax import lax
from jax.experimental import pallas as pl
from jax.experimental.pallas import tpu as pltpu
from jax.experimental.pallas import tpu_sc as plsc

N = 10000
D = 128
E = 320000

NC = 2
NS = 16
DH = D // NC
CH = 128
EPT = 20480
E_PAD = EPT * NS
NCHUNK = EPT // CH
NT = 10240
RPT = NT // NS
CW = 16


def _make_sc_agg(with_cnt: bool):
  out_types = [jax.ShapeDtypeStruct((NC, NT, DH), jnp.float32)]
  if with_cnt:
    out_types.append(jax.ShapeDtypeStruct((NT, CW), jnp.float32))
  scratch = [
      pltpu.VMEM_SHARED((NT, DH), jnp.float32),
      pltpu.VMEM((NCHUNK, CH), jnp.int32),
      pltpu.VMEM((NCHUNK, CH), jnp.int32),
      pltpu.VMEM((2, CH, DH), jnp.float32),
      pltpu.SemaphoreType.DMA,
      pltpu.SemaphoreType.DMA,
  ]
  if with_cnt:
    scratch += [
        pltpu.VMEM_SHARED((NT, CW), jnp.float32),
        pltpu.VMEM((RPT, CW), jnp.float32),
        pltpu.VMEM((CH, CW), jnp.float32),
    ]
  mesh = plsc.VectorSubcoreMesh(core_axis_name="c", subcore_axis_name="s")

  def body(ya_hbm, yb_hbm, src_hbm, dst_hbm, *rest):
    if with_cnt:
      (agg_out, cnt_out, acc, sidx, didx, rows, sem0, sem1,
       cnt, zbc, ones) = rest
    else:
      agg_out, acc, sidx, didx, rows, sem0, sem1 = rest
      cnt = zbc = ones = cnt_out = None
    c = lax.axis_index("c")
    s = lax.axis_index("s")

    pltpu.sync_copy(src_hbm.at[s], sidx)
    pltpu.sync_copy(dst_hbm.at[s], didx)

    zv = jnp.zeros((16,), jnp.float32)

    def fill_zb(i, carry):
      r = i // (DH // 16)
      q = i % (DH // 16)
      rows[0, r, pl.ds(q * 16, 16)] = zv
      return carry

    lax.fori_loop(0, CH * (DH // 16), fill_zb, None)
    if with_cnt:
      ov = jnp.ones((16,), jnp.float32)

      def fill_cnt(i, carry):
        zbc[i, :] = zv

        @pl.when(i < CH)
        def _():
          ones[i, :] = ov

        return carry

      lax.fori_loop(0, RPT, fill_cnt, None)

    for k in range(RPT // CH):
      pltpu.sync_copy(rows.at[0], acc.at[pl.ds(s * RPT + k * CH, CH)])
    if with_cnt:
      @pl.when(c == 0)
      def _():
        pltpu.sync_copy(zbc, cnt.at[pl.ds(s * RPT, RPT)])
    plsc.subcore_barrier()

    def start_gather(chunk, slot, sem):
      @pl.when(c == 0)
      def _():
        pltpu.async_copy(ya_hbm.at[sidx.at[chunk]], rows.at[slot], sem)

      @pl.when(c == 1)
      def _():
        pltpu.async_copy(yb_hbm.at[sidx.at[chunk]], rows.at[slot], sem)

    def wait_gather(slot, sem):
      pltpu.make_async_copy(ya_hbm.at[sidx.at[0]], rows.at[slot], sem).wait()

    def scatter(chunk, slot):
      pltpu.sync_copy(rows.at[slot], acc.at[didx.at[chunk]], add=True)
      if with_cnt:
        @pl.when(c == 0)
        def _():
          pltpu.sync_copy(ones, cnt.at[didx.at[chunk]], add=True)

    start_gather(0, 0, sem0)

    def outer(g, carry):
      c0 = g * 2
      start_gather(c0 + 1, 1, sem1)
      wait_gather(0, sem0)
      scatter(c0, 0)

      @pl.when(c0 + 2 < NCHUNK)
      def _():
        start_gather(c0 + 2, 0, sem0)

      wait_gather(1, sem1)
      scatter(c0 + 1, 1)
      return carry

    lax.fori_loop(0, NCHUNK // 2, outer, None)
    plsc.subcore_barrier()

    pltpu.sync_copy(acc.at[pl.ds(s * RPT, RPT)],
                    agg_out.at[c, pl.ds(s * RPT, RPT)])
    if with_cnt:
      @pl.when(c == 0)
      def _():
        pltpu.sync_copy(cnt.at[pl.ds(s * RPT, RPT)],
                        cnt_out.at[pl.ds(s * RPT, RPT)])

  return pl.kernel(
      body,
      out_type=tuple(out_types) if with_cnt else out_types[0],
      mesh=mesh,
      scratch_types=scratch,
      compiler_params=pltpu.CompilerParams(use_tc_tiling_on_sc=False),
  )


_sc_agg_cnt = _make_sc_agg(True)
_sc_agg = _make_sc_agg(False)

BM = 1000


def _tc_pre_body(x_ref, wl_ref, wr_ref, b_ref, ya_ref, yb_ref, r_ref):
  xb = x_ref[...]
  y = jnp.dot(xb, wl_ref[...], preferred_element_type=jnp.float32)
  ya_ref[...] = y[:, :DH]
  yb_ref[...] = y[:, DH:]
  r_ref[...] = (jnp.dot(xb, wr_ref[...], preferred_element_type=jnp.float32)
                + b_ref[...])


_tc_pre = pl.pallas_call(
    _tc_pre_body,
    grid=(N // BM,),
    in_specs=[
        pl.BlockSpec((BM, D), lambda i: (i, 0)),
        pl.BlockSpec((D, D), lambda i: (0, 0)),
        pl.BlockSpec((D, D), lambda i: (0, 0)),
        pl.BlockSpec((1, D), lambda i: (0, 0)),
    ],
    out_specs=[
        pl.BlockSpec((BM, DH), lambda i: (i, 0)),
        pl.BlockSpec((BM, DH), lambda i: (i, 0)),
        pl.BlockSpec((BM, D), lambda i: (i, 0)),
    ],
    out_shape=[
        jax.ShapeDtypeStruct((N, DH), jnp.float32),
        jax.ShapeDtypeStruct((N, DH), jnp.float32),
        jax.ShapeDtypeStruct((N, D), jnp.float32),
    ],
)


def _mean_block(aggp_ref, cnt_ref):
  agg = jnp.concatenate([aggp_ref[0], aggp_ref[1]], axis=-1)
  inv = 1.0 / jnp.maximum(cnt_ref[:, 0:1], 1.0)
  return agg * inv


def _tc_mid_body(aggp_ref, cnt_ref, r1_ref, wl_ref, wr_ref, b_ref,
                 ya_ref, yb_ref, r2_ref):
  h = jnp.maximum(_mean_block(aggp_ref, cnt_ref) + r1_ref[...], 0.0)
  y = jnp.dot(h, wl_ref[...], preferred_element_type=jnp.float32)
  ya_ref[...] = y[:, :DH]
  yb_ref[...] = y[:, DH:]
  r2_ref[...] = (jnp.dot(h, wr_ref[...], preferred_element_type=jnp.float32)
                 + b_ref[...])


_tc_mid = pl.pallas_call(
    _tc_mid_body,
    grid=(N // BM,),
    in_specs=[
        pl.BlockSpec((NC, BM, DH), lambda i: (0, i, 0)),
        pl.BlockSpec((BM, CW), lambda i: (i, 0)),
        pl.BlockSpec((BM, D), lambda i: (i, 0)),
        pl.BlockSpec((D, D), lambda i: (0, 0)),
        pl.BlockSpec((D, D), lambda i: (0, 0)),
        pl.BlockSpec((1, D), lambda i: (0, 0)),
    ],
    out_specs=[
        pl.BlockSpec((BM, DH), lambda i: (i, 0)),
        pl.BlockSpec((BM, DH), lambda i: (i, 0)),
        pl.BlockSpec((BM, D), lambda i: (i, 0)),
    ],
    out_shape=[
        jax.ShapeDtypeStruct((N, DH), jnp.float32),
        jax.ShapeDtypeStruct((N, DH), jnp.float32),
        jax.ShapeDtypeStruct((N, D), jnp.float32),
    ],
)


def _tc_fin_body(aggp_ref, cnt_ref, r2_ref, out_ref):
  out_ref[...] = _mean_block(aggp_ref, cnt_ref) + r2_ref[...]


_tc_fin = pl.pallas_call(
    _tc_fin_body,
    grid=(N // BM,),
    in_specs=[
        pl.BlockSpec((NC, BM, DH), lambda i: (0, i, 0)),
        pl.BlockSpec((BM, CW), lambda i: (i, 0)),
        pl.BlockSpec((BM, D), lambda i: (i, 0)),
    ],
    out_specs=pl.BlockSpec((BM, D), lambda i: (i, 0)),
    out_shape=jax.ShapeDtypeStruct((N, D), jnp.float32),
)


def kernel(x, edge_index, W1_l, b1, W1_r, W2_l, b2, W2_r):
  ei = edge_index.astype(jnp.int32)
  src = ei[0]
  dst = ei[1]
  pad = E_PAD - E
  pad_idx = jnp.arange(pad, dtype=jnp.int32)
  src_p = jnp.concatenate([src, pad_idx % N])
  dst_p = jnp.concatenate([dst, N + pad_idx % (NT - N)])
  src3 = src_p.reshape(NS, NCHUNK, CH)
  dst3 = dst_p.reshape(NS, NCHUNK, CH)

  b1r = b1.reshape(1, D)
  b2r = b2.reshape(1, D)

  y1a, y1b, r1 = _tc_pre(x, W1_l, W1_r, b1r)
  aggp1, cnt = _sc_agg_cnt(y1a, y1b, src3, dst3)
  y2a, y2b, r2 = _tc_mid(aggp1, cnt, r1, W2_l, W2_r, b2r)
  aggp2 = _sc_agg(y2a, y2b, src3, dst3)
  return _tc_fin(aggp2, cnt, r2)


# speedup vs baseline: 12.2934x; 1.1760x over previous
import functools

import jax
import jax.numpy as jnp
from jax import lax
from jax.experimental import pallas as pl
from jax.experimental.pallas import tpu as pltpu
from jax.experimental.pallas import tpu_sc as plsc

N = 10000
D = 128
E = 320000

NC = 2
NS = 16
DH = D // NC
CH = 128
EPT = 20480
E_PAD = EPT * NS
NCHUNK = EPT // CH
NPASS = 2
HALF = NCHUNK // NPASS
NB = 4
NT = 10240
RPT = NT // NS
CW = 16


def _make_sc_agg(with_cnt: bool):
  out_types = [jax.ShapeDtypeStruct((NC, NT, DH), jnp.float32)]
  if with_cnt:
    out_types.append(jax.ShapeDtypeStruct((NT, CW), jnp.float32))
  scratch = [
      pltpu.VMEM_SHARED((NT, DH), jnp.float32),
      pltpu.VMEM((HALF, CH), jnp.int32),
      pltpu.VMEM((HALF, CH), jnp.int32),
      pltpu.VMEM((NB, CH, DH), jnp.float32),
      [pltpu.SemaphoreType.DMA] * NB,
      [pltpu.SemaphoreType.DMA] * NB,
      pltpu.SemaphoreType.DMA,
  ]
  if with_cnt:
    scratch += [
        pltpu.VMEM_SHARED((NT, CW), jnp.float32),
        pltpu.VMEM((RPT, CW), jnp.float32),
        pltpu.VMEM((CH, CW), jnp.float32),
    ]
  mesh = plsc.VectorSubcoreMesh(core_axis_name="c", subcore_axis_name="s")

  def body(ya_hbm, yb_hbm, src_hbm, dst_hbm, *rest):
    if with_cnt:
      (agg_out, cnt_out, acc, sidx, didx, rows, sg, ss, semc,
       cnt, zbc, ones) = rest
    else:
      agg_out, acc, sidx, didx, rows, sg, ss, semc = rest
      cnt = zbc = ones = cnt_out = None
    c = lax.axis_index("c")
    s = lax.axis_index("s")

    zv = jnp.zeros((16,), jnp.float32)

    def fill_zb(i, carry):
      r = i // (DH // 16)
      q = i % (DH // 16)
      rows[0, r, pl.ds(q * 16, 16)] = zv
      return carry

    lax.fori_loop(0, CH * (DH // 16), fill_zb, None)
    if with_cnt:
      ov = jnp.ones((16,), jnp.float32)

      def fill_cnt(i, carry):
        zbc[i, :] = zv

        @pl.when(i < CH)
        def _():
          ones[i, :] = ov

        return carry

      lax.fori_loop(0, RPT, fill_cnt, None)

    for k in range(RPT // CH):
      pltpu.sync_copy(rows.at[0], acc.at[pl.ds(s * RPT + k * CH, CH)])
    if with_cnt:
      pltpu.sync_copy(zbc, cnt.at[pl.ds(s * RPT, RPT)])
    plsc.subcore_barrier()

    def start_gather(chunk, slot):
      @pl.when(c == 0)
      def _():
        pltpu.async_copy(ya_hbm.at[sidx.at[chunk]], rows.at[slot], sg[slot])

      @pl.when(c == 1)
      def _():
        pltpu.async_copy(yb_hbm.at[sidx.at[chunk]], rows.at[slot], sg[slot])

    def wait_gather(slot):
      pltpu.make_async_copy(
          ya_hbm.at[sidx.at[0]], rows.at[slot], sg[slot]).wait()

    def start_scatter(chunk, slot):
      pltpu.async_copy(rows.at[slot], acc.at[didx.at[chunk]], ss[slot],
                       add=True)

    def wait_scatter(slot):
      pltpu.make_async_copy(
          rows.at[slot], acc.at[didx.at[0]], ss[slot]).wait()

    def start_cnt(chunk):
      pltpu.async_copy(ones, cnt.at[didx.at[chunk]], semc, add=True)

    def wait_cnt():
      pltpu.make_async_copy(ones, cnt.at[didx.at[0]], semc).wait()

    for p in range(NPASS):
      pltpu.sync_copy(src_hbm.at[s, p], sidx)
      pltpu.sync_copy(dst_hbm.at[s, p], didx)
      for b in range(NB):
        start_gather(b, b)

      def body4(g, carry):
        for b in range(NB):
          i = g * NB + b
          wait_gather(b)
          start_scatter(i, b)
          if with_cnt:
            @pl.when(i > 0)
            def _():
              wait_cnt()

            start_cnt(i)

          @pl.when(i + NB < HALF)
          def _():
            wait_scatter(b)
            start_gather(i + NB, b)

        return carry

      lax.fori_loop(0, HALF // NB, body4, None)
      for b in range(NB):
        wait_scatter(b)
      if with_cnt:
        wait_cnt()
    plsc.subcore_barrier()

    pltpu.sync_copy(acc.at[pl.ds(s * RPT, RPT)],
                    agg_out.at[c, pl.ds(s * RPT, RPT)])
    if with_cnt:
      @pl.when(c == 0)
      def _():
        pltpu.sync_copy(cnt.at[pl.ds(s * RPT, RPT)],
                        cnt_out.at[pl.ds(s * RPT, RPT)])

  return pl.kernel(
      body,
      out_type=tuple(out_types) if with_cnt else out_types[0],
      mesh=mesh,
      scratch_types=scratch,
      compiler_params=pltpu.CompilerParams(use_tc_tiling_on_sc=False),
  )


_sc_agg_cnt = _make_sc_agg(True)
_sc_agg = _make_sc_agg(False)

BM = 1000


def _tc_pre_body(x_ref, wl_ref, wr_ref, b_ref, ya_ref, yb_ref, r_ref):
  xb = x_ref[...]
  y = jnp.dot(xb, wl_ref[...], preferred_element_type=jnp.float32)
  ya_ref[...] = y[:, :DH]
  yb_ref[...] = y[:, DH:]
  r_ref[...] = (jnp.dot(xb, wr_ref[...], preferred_element_type=jnp.float32)
                + b_ref[...])


_tc_pre = pl.pallas_call(
    _tc_pre_body,
    grid=(N // BM,),
    in_specs=[
        pl.BlockSpec((BM, D), lambda i: (i, 0)),
        pl.BlockSpec((D, D), lambda i: (0, 0)),
        pl.BlockSpec((D, D), lambda i: (0, 0)),
        pl.BlockSpec((1, D), lambda i: (0, 0)),
    ],
    out_specs=[
        pl.BlockSpec((BM, DH), lambda i: (i, 0)),
        pl.BlockSpec((BM, DH), lambda i: (i, 0)),
        pl.BlockSpec((BM, D), lambda i: (i, 0)),
    ],
    out_shape=[
        jax.ShapeDtypeStruct((N, DH), jnp.float32),
        jax.ShapeDtypeStruct((N, DH), jnp.float32),
        jax.ShapeDtypeStruct((N, D), jnp.float32),
    ],
)


def _mean_block(aggp_ref, cnt_ref):
  agg = jnp.concatenate([aggp_ref[0], aggp_ref[1]], axis=-1)
  inv = 1.0 / jnp.maximum(cnt_ref[:, 0:1], 1.0)
  return agg * inv


def _tc_mid_body(aggp_ref, cnt_ref, r1_ref, wl_ref, wr_ref, b_ref,
                 ya_ref, yb_ref, r2_ref):
  h = jnp.maximum(_mean_block(aggp_ref, cnt_ref) + r1_ref[...], 0.0)
  y = jnp.dot(h, wl_ref[...], preferred_element_type=jnp.float32)
  ya_ref[...] = y[:, :DH]
  yb_ref[...] = y[:, DH:]
  r2_ref[...] = (jnp.dot(h, wr_ref[...], preferred_element_type=jnp.float32)
                 + b_ref[...])


_tc_mid = pl.pallas_call(
    _tc_mid_body,
    grid=(N // BM,),
    in_specs=[
        pl.BlockSpec((NC, BM, DH), lambda i: (0, i, 0)),
        pl.BlockSpec((BM, CW), lambda i: (i, 0)),
        pl.BlockSpec((BM, D), lambda i: (i, 0)),
        pl.BlockSpec((D, D), lambda i: (0, 0)),
        pl.BlockSpec((D, D), lambda i: (0, 0)),
        pl.BlockSpec((1, D), lambda i: (0, 0)),
    ],
    out_specs=[
        pl.BlockSpec((BM, DH), lambda i: (i, 0)),
        pl.BlockSpec((BM, DH), lambda i: (i, 0)),
        pl.BlockSpec((BM, D), lambda i: (i, 0)),
    ],
    out_shape=[
        jax.ShapeDtypeStruct((N, DH), jnp.float32),
        jax.ShapeDtypeStruct((N, DH), jnp.float32),
        jax.ShapeDtypeStruct((N, D), jnp.float32),
    ],
)


def _tc_fin_body(aggp_ref, cnt_ref, r2_ref, out_ref):
  out_ref[...] = _mean_block(aggp_ref, cnt_ref) + r2_ref[...]


_tc_fin = pl.pallas_call(
    _tc_fin_body,
    grid=(N // BM,),
    in_specs=[
        pl.BlockSpec((NC, BM, DH), lambda i: (0, i, 0)),
        pl.BlockSpec((BM, CW), lambda i: (i, 0)),
        pl.BlockSpec((BM, D), lambda i: (i, 0)),
    ],
    out_specs=pl.BlockSpec((BM, D), lambda i: (i, 0)),
    out_shape=jax.ShapeDtypeStruct((N, D), jnp.float32),
)


def kernel(x, edge_index, W1_l, b1, W1_r, W2_l, b2, W2_r):
  ei = edge_index.astype(jnp.int32)
  src = ei[0]
  dst = ei[1]
  pad = E_PAD - E
  pad_idx = jnp.arange(pad, dtype=jnp.int32)
  src_p = jnp.concatenate([src, pad_idx % N])
  dst_p = jnp.concatenate([dst, N + pad_idx % (NT - N)])
  src3 = src_p.reshape(NS, NPASS, HALF, CH)
  dst3 = dst_p.reshape(NS, NPASS, HALF, CH)

  b1r = b1.reshape(1, D)
  b2r = b2.reshape(1, D)

  y1a, y1b, r1 = _tc_pre(x, W1_l, W1_r, b1r)
  aggp1, cnt = _sc_agg_cnt(y1a, y1b, src3, dst3)
  y2a, y2b, r2 = _tc_mid(aggp1, cnt, r1, W2_l, W2_r, b2r)
  aggp2 = _sc_agg(y2a, y2b, src3, dst3)
  return _tc_fin(aggp2, cnt, r2)


# speedup vs baseline: 12.4069x; 1.0092x over previous
import functools

import jax
import jax.numpy as jnp
from jax import lax
from jax.experimental import pallas as pl
from jax.experimental.pallas import tpu as pltpu
from jax.experimental.pallas import tpu_sc as plsc

N = 10000
D = 128
E = 320000

NC = 2
NS = 16
DH = D // NC
CH = 128
EPT = 20480
E_PAD = EPT * NS
NCHUNK = EPT // CH
NPASS = 2
HALF = NCHUNK // NPASS
NB = 4
NT = 10240
RPT = NT // NS
CW = 16


def _make_sc_agg(with_cnt: bool):
  out_types = [jax.ShapeDtypeStruct((N, D), jnp.float32)]
  if with_cnt:
    out_types.append(jax.ShapeDtypeStruct((NT, CW), jnp.float32))
  scratch = [
      pltpu.VMEM_SHARED((NT, DH), jnp.float32),
      pltpu.VMEM((HALF, CH), jnp.int32),
      pltpu.VMEM((HALF, CH), jnp.int32),
      pltpu.VMEM((NB, CH, DH), jnp.float32),
      [pltpu.SemaphoreType.DMA] * NB,
      [pltpu.SemaphoreType.DMA] * NB,
      pltpu.SemaphoreType.DMA,
      pltpu.VMEM((CH, CW), jnp.float32),
  ]
  if with_cnt:
    scratch += [
        pltpu.VMEM_SHARED((NT, CW), jnp.float32),
        pltpu.VMEM((RPT, CW), jnp.float32),
        pltpu.VMEM((CH, CW), jnp.float32),
    ]
  mesh = plsc.VectorSubcoreMesh(core_axis_name="c", subcore_axis_name="s")

  def body(ya_hbm, yb_hbm, src_hbm, dst_hbm, r_hbm, *rest):
    if with_cnt:
      (out_hbm, cnt_out, acc, sidx, didx, rows, sg, ss, semc, cbuf,
       cnt, zbc, ones) = rest
      cnt_hbm = None
    else:
      cnt_hbm = rest[0]
      out_hbm, acc, sidx, didx, rows, sg, ss, semc, cbuf = rest[1:]
      cnt = zbc = ones = cnt_out = None
    c = lax.axis_index("c")
    s = lax.axis_index("s")

    zv = jnp.zeros((16,), jnp.float32)

    def fill_zb(i, carry):
      r = i // (DH // 16)
      q = i % (DH // 16)
      rows[0, r, pl.ds(q * 16, 16)] = zv
      return carry

    lax.fori_loop(0, CH * (DH // 16), fill_zb, None)
    if with_cnt:
      ov = jnp.ones((16,), jnp.float32)

      def fill_cnt(i, carry):
        zbc[i, :] = zv

        @pl.when(i < CH)
        def _():
          ones[i, :] = ov

        return carry

      lax.fori_loop(0, RPT, fill_cnt, None)

    for k in range(RPT // CH):
      pltpu.sync_copy(rows.at[0], acc.at[pl.ds(s * RPT + k * CH, CH)])
    if with_cnt:
      pltpu.sync_copy(zbc, cnt.at[pl.ds(s * RPT, RPT)])
    plsc.subcore_barrier()

    def start_gather(chunk, slot):
      @pl.when(c == 0)
      def _():
        pltpu.async_copy(ya_hbm.at[sidx.at[chunk]], rows.at[slot], sg[slot])

      @pl.when(c == 1)
      def _():
        pltpu.async_copy(yb_hbm.at[sidx.at[chunk]], rows.at[slot], sg[slot])

    def wait_gather(slot):
      pltpu.make_async_copy(
          ya_hbm.at[sidx.at[0]], rows.at[slot], sg[slot]).wait()

    def start_scatter(chunk, slot):
      pltpu.async_copy(rows.at[slot], acc.at[didx.at[chunk]], ss[slot],
                       add=True)

    def wait_scatter(slot):
      pltpu.make_async_copy(
          rows.at[slot], acc.at[didx.at[0]], ss[slot]).wait()

    def start_cnt(chunk):
      pltpu.async_copy(ones, cnt.at[didx.at[chunk]], semc, add=True)

    def wait_cnt():
      pltpu.make_async_copy(ones, cnt.at[didx.at[0]], semc).wait()

    for p in range(NPASS):
      pltpu.sync_copy(src_hbm.at[s, p], sidx)
      pltpu.sync_copy(dst_hbm.at[s, p], didx)
      for b in range(NB):
        start_gather(b, b)

      def body4(g, carry):
        for b in range(NB):
          i = g * NB + b
          wait_gather(b)
          start_scatter(i, b)
          if with_cnt:
            @pl.when(i > 0)
            def _():
              wait_cnt()

            start_cnt(i)

          @pl.when(i + NB < HALF)
          def _():
            wait_scatter(b)
            start_gather(i + NB, b)

        return carry

      lax.fori_loop(0, HALF // NB, body4, None)
      for b in range(NB):
        wait_scatter(b)
      if with_cnt:
        wait_cnt()
    plsc.subcore_barrier()

    col = c * DH

    def epi_chunk(rb, nr):
      pltpu.sync_copy(acc.at[pl.ds(rb, nr)], rows.at[0, pl.ds(0, nr)])
      pltpu.sync_copy(r_hbm.at[pl.ds(rb, nr), pl.ds(col, DH)],
                      rows.at[1, pl.ds(0, nr)])
      if with_cnt:
        pltpu.sync_copy(cnt.at[pl.ds(rb, nr)], cbuf.at[pl.ds(0, nr)])
      else:
        pltpu.sync_copy(cnt_hbm.at[pl.ds(rb, nr)], cbuf.at[pl.ds(0, nr)])

      def epi_row(rr, carry):
        inv = 1.0 / jnp.maximum(cbuf[rr, :], 1.0)
        for q in range(DH // 16):
          v = rows[0, rr, pl.ds(q * 16, 16)] * inv
          v = v + rows[1, rr, pl.ds(q * 16, 16)]
          if with_cnt:
            v = jnp.maximum(v, 0.0)
          rows[0, rr, pl.ds(q * 16, 16)] = v
        return carry

      lax.fori_loop(0, nr, epi_row, None)
      pltpu.sync_copy(rows.at[0, pl.ds(0, nr)],
                      out_hbm.at[pl.ds(rb, nr), pl.ds(col, DH)])

    @pl.when(s < NS - 1)
    def _():
      for k in range(RPT // CH):
        epi_chunk(s * RPT + k * CH, CH)

    @pl.when(s == NS - 1)
    def _():
      base = (NS - 1) * RPT
      nfull = (N - base) // CH
      for k in range(nfull):
        epi_chunk(base + k * CH, CH)
      tail = N - base - nfull * CH
      if tail:
        epi_chunk(base + nfull * CH, tail)

    if with_cnt:
      @pl.when(c == 0)
      def _():
        pltpu.sync_copy(cnt.at[pl.ds(s * RPT, RPT)],
                        cnt_out.at[pl.ds(s * RPT, RPT)])

  return pl.kernel(
      body,
      out_type=tuple(out_types) if with_cnt else out_types[0],
      mesh=mesh,
      scratch_types=scratch,
      compiler_params=pltpu.CompilerParams(use_tc_tiling_on_sc=False),
  )


_sc_agg_cnt = _make_sc_agg(True)
_sc_agg = _make_sc_agg(False)

BM = 1000


def _tc_pre_body(x_ref, wl_ref, wr_ref, b_ref, ya_ref, yb_ref, r_ref):
  xb = x_ref[...]
  y = jnp.dot(xb, wl_ref[...], preferred_element_type=jnp.float32)
  ya_ref[...] = y[:, :DH]
  yb_ref[...] = y[:, DH:]
  r_ref[...] = (jnp.dot(xb, wr_ref[...], preferred_element_type=jnp.float32)
                + b_ref[...])


_tc_pre = pl.pallas_call(
    _tc_pre_body,
    grid=(N // BM,),
    in_specs=[
        pl.BlockSpec((BM, D), lambda i: (i, 0)),
        pl.BlockSpec((D, D), lambda i: (0, 0)),
        pl.BlockSpec((D, D), lambda i: (0, 0)),
        pl.BlockSpec((1, D), lambda i: (0, 0)),
    ],
    out_specs=[
        pl.BlockSpec((BM, DH), lambda i: (i, 0)),
        pl.BlockSpec((BM, DH), lambda i: (i, 0)),
        pl.BlockSpec((BM, D), lambda i: (i, 0)),
    ],
    out_shape=[
        jax.ShapeDtypeStruct((N, DH), jnp.float32),
        jax.ShapeDtypeStruct((N, DH), jnp.float32),
        jax.ShapeDtypeStruct((N, D), jnp.float32),
    ],
)


def kernel(x, edge_index, W1_l, b1, W1_r, W2_l, b2, W2_r):
  ei = edge_index.astype(jnp.int32)
  src = ei[0]
  dst = ei[1]
  pad = E_PAD - E
  pad_idx = jnp.arange(pad, dtype=jnp.int32)
  src_p = jnp.concatenate([src, pad_idx % N])
  dst_p = jnp.concatenate([dst, N + pad_idx % (NT - N)])
  src3 = src_p.reshape(NS, NPASS, HALF, CH)
  dst3 = dst_p.reshape(NS, NPASS, HALF, CH)

  b1r = b1.reshape(1, D)
  b2r = b2.reshape(1, D)

  y1a, y1b, r1 = _tc_pre(x, W1_l, W1_r, b1r)
  h, cnt = _sc_agg_cnt(y1a, y1b, src3, dst3, r1)
  y2a, y2b, r2 = _tc_pre(h, W2_l, W2_r, b2r)
  return _sc_agg(y2a, y2b, src3, dst3, r2, cnt)


# speedup vs baseline: 13.0824x; 1.0544x over previous
import functools

import jax
import jax.numpy as jnp
from jax import lax
from jax.experimental import pallas as pl
from jax.experimental.pallas import tpu as pltpu
from jax.experimental.pallas import tpu_sc as plsc

N = 10000
D = 128
E = 320000

NC = 2
NS = 16
DH = D // NC
CH = 128
EPT = 20480
E_PAD = EPT * NS
NCHUNK = EPT // CH
NPASS = 2
HALF = NCHUNK // NPASS
NB = 4
NT = 10240
RPT = NT // NS
CW = 16
NTR = NT // CW


def _make_sc_agg(with_cnt: bool):
  out_types = [jax.ShapeDtypeStruct((N, D), jnp.float32)]
  if with_cnt:
    out_types.append(jax.ShapeDtypeStruct((NTR, CW), jnp.float32))
  scratch = [
      pltpu.VMEM_SHARED((NT, DH), jnp.float32),
      pltpu.VMEM((HALF, CH), jnp.int32),
      pltpu.VMEM((HALF, CH), jnp.int32),
      pltpu.VMEM((NB, CH, DH), jnp.float32),
      [pltpu.SemaphoreType.DMA] * NB,
      [pltpu.SemaphoreType.DMA] * NB,
      pltpu.VMEM((CH, DH), jnp.float32),
      pltpu.VMEM((CH // CW, CW), jnp.float32),
  ]
  if with_cnt:
    scratch += [
        pltpu.VMEM_SHARED((NTR, CW), jnp.float32),
        pltpu.VMEM((NTR, CW), jnp.float32),
        pltpu.VMEM((NTR // CH, CH), jnp.int32),
        pltpu.VMEM((NTR // NS, CW), jnp.float32),
    ]
  mesh = plsc.VectorSubcoreMesh(core_axis_name="c", subcore_axis_name="s")

  def body(ya_hbm, yb_hbm, src_hbm, dst_hbm, r_hbm, *rest):
    if with_cnt:
      (out_hbm, cnt_out, acc, sidx, didx, rows, sg, ss, zb, cflat,
       cnt, hist, riota, zbc) = rest
      cnt_hbm = None
    else:
      cnt_hbm = rest[0]
      out_hbm, acc, sidx, didx, rows, sg, ss, zb, cflat = rest[1:]
      cnt = hist = riota = zbc = cnt_out = None
    c = lax.axis_index("c")
    s = lax.axis_index("s")

    def start_gather(chunk, slot):
      @pl.when(c == 0)
      def _():
        pltpu.async_copy(ya_hbm.at[sidx.at[chunk]], rows.at[slot], sg[slot])

      @pl.when(c == 1)
      def _():
        pltpu.async_copy(yb_hbm.at[sidx.at[chunk]], rows.at[slot], sg[slot])

    def wait_gather(slot):
      pltpu.make_async_copy(
          ya_hbm.at[sidx.at[0]], rows.at[slot], sg[slot]).wait()

    def start_scatter(chunk, slot):
      pltpu.async_copy(rows.at[slot], acc.at[didx.at[chunk]], ss[slot],
                       add=True)

    def wait_scatter(slot):
      pltpu.make_async_copy(
          rows.at[slot], acc.at[didx.at[0]], ss[slot]).wait()

    pltpu.sync_copy(src_hbm.at[s, 0], sidx)
    pltpu.sync_copy(dst_hbm.at[s, 0], didx)
    for b in range(NB):
      start_gather(b, b)

    zv = jnp.zeros((16,), jnp.float32)

    def fill_zb(i, carry):
      r = i // (DH // 16)
      q = i % (DH // 16)
      zb[r, pl.ds(q * 16, 16)] = zv
      return carry

    lax.fori_loop(0, CH * (DH // 16), fill_zb, None)
    if with_cnt:
      lanes = lax.iota(jnp.int32, 16)

      def fill_cnt(i, carry):
        hist[i, :] = zv

        @pl.when(i < NTR // NS)
        def _():
          zbc[i, :] = zv

        @pl.when(i < (NTR // CH) * (CH // 16))
        def _():
          riota[i // (CH // 16), pl.ds((i % (CH // 16)) * 16, 16)] = (
              lanes + i * 16)

        return carry

      lax.fori_loop(0, NTR, fill_cnt, None)

    for k in range(RPT // CH):
      pltpu.sync_copy(zb, acc.at[pl.ds(s * RPT + k * CH, CH)])
    if with_cnt:
      pltpu.sync_copy(zbc, cnt.at[pl.ds(s * (NTR // NS), NTR // NS)])
    plsc.subcore_barrier()

    ones_v = jnp.ones((16,), jnp.float32)
    for p in range(NPASS):
      if p > 0:
        pltpu.sync_copy(src_hbm.at[s, p], sidx)
        pltpu.sync_copy(dst_hbm.at[s, p], didx)
        for b in range(NB):
          start_gather(b, b)

      def body4(g, carry):
        for b in range(NB):
          i = g * NB + b
          wait_gather(b)
          start_scatter(i, b)
          if with_cnt:
            for q in range(CH // 16):
              dv = didx[i, pl.ds(q * 16, 16)]
              plsc.addupdate_scatter(
                  hist,
                  [lax.shift_right_logical(dv, 4),
                   lax.bitwise_and(dv, CW - 1)],
                  ones_v)

          @pl.when(i + NB < HALF)
          def _():
            wait_scatter(b)
            start_gather(i + NB, b)

        return carry

      lax.fori_loop(0, HALF // NB, body4, None)
      for b in range(NB):
        wait_scatter(b)

    if with_cnt:
      for k in range(NTR // CH):
        pltpu.sync_copy(hist.at[pl.ds(k * CH, CH)],
                        cnt.at[riota.at[k]], add=True)
    plsc.subcore_barrier()

    col = c * DH

    def epi_chunk(rb, nr):
      pltpu.sync_copy(acc.at[pl.ds(rb, nr)], rows.at[0, pl.ds(0, nr)])
      pltpu.sync_copy(r_hbm.at[pl.ds(rb, nr), pl.ds(col, DH)],
                      rows.at[1, pl.ds(0, nr)])
      cnt_src = cnt if with_cnt else cnt_hbm
      pltpu.sync_copy(cnt_src.at[pl.ds(rb // CW, nr // CW)],
                      cflat.at[pl.ds(0, nr // CW)])

      def epi_row(rr, carry):
        cs = plsc.load_gather(
            cflat, [jnp.full((16,), rr // CW, jnp.int32),
                    jnp.full((16,), rr % CW, jnp.int32)])
        inv = 1.0 / jnp.maximum(cs, 1.0)
        for q in range(DH // 16):
          v = rows[0, rr, pl.ds(q * 16, 16)] * inv
          v = v + rows[1, rr, pl.ds(q * 16, 16)]
          if with_cnt:
            v = jnp.maximum(v, 0.0)
          rows[0, rr, pl.ds(q * 16, 16)] = v
        return carry

      lax.fori_loop(0, nr, epi_row, None)
      pltpu.sync_copy(rows.at[0, pl.ds(0, nr)],
                      out_hbm.at[pl.ds(rb, nr), pl.ds(col, DH)])

    @pl.when(s < NS - 1)
    def _():
      for k in range(RPT // CH):
        epi_chunk(s * RPT + k * CH, CH)

    @pl.when(s == NS - 1)
    def _():
      base = (NS - 1) * RPT
      nfull = (N - base) // CH
      for k in range(nfull):
        epi_chunk(base + k * CH, CH)
      tail = N - base - nfull * CH
      if tail:
        epi_chunk(base + nfull * CH, tail)

    if with_cnt:
      @pl.when(c == 0)
      def _():
        pltpu.sync_copy(cnt.at[pl.ds(s * (NTR // NS), NTR // NS)],
                        cnt_out.at[pl.ds(s * (NTR // NS), NTR // NS)])

  return pl.kernel(
      body,
      out_type=tuple(out_types) if with_cnt else out_types[0],
      mesh=mesh,
      scratch_types=scratch,
      compiler_params=pltpu.CompilerParams(use_tc_tiling_on_sc=False,
                                           needs_layout_passes=False),
  )


_sc_agg_cnt = _make_sc_agg(True)
_sc_agg = _make_sc_agg(False)

BM = 1000


def _tc_pre_body(x_ref, wl_ref, wr_ref, b_ref, ya_ref, yb_ref, r_ref):
  xb = x_ref[...]
  y = jnp.dot(xb, wl_ref[...], preferred_element_type=jnp.float32)
  ya_ref[...] = y[:, :DH]
  yb_ref[...] = y[:, DH:]
  r_ref[...] = (jnp.dot(xb, wr_ref[...], preferred_element_type=jnp.float32)
                + b_ref[...])


_tc_pre = pl.pallas_call(
    _tc_pre_body,
    grid=(N // BM,),
    in_specs=[
        pl.BlockSpec((BM, D), lambda i: (i, 0)),
        pl.BlockSpec((D, D), lambda i: (0, 0)),
        pl.BlockSpec((D, D), lambda i: (0, 0)),
        pl.BlockSpec((1, D), lambda i: (0, 0)),
    ],
    out_specs=[
        pl.BlockSpec((BM, DH), lambda i: (i, 0)),
        pl.BlockSpec((BM, DH), lambda i: (i, 0)),
        pl.BlockSpec((BM, D), lambda i: (i, 0)),
    ],
    out_shape=[
        jax.ShapeDtypeStruct((N, DH), jnp.float32),
        jax.ShapeDtypeStruct((N, DH), jnp.float32),
        jax.ShapeDtypeStruct((N, D), jnp.float32),
    ],
)


def kernel(x, edge_index, W1_l, b1, W1_r, W2_l, b2, W2_r):
  ei = edge_index.astype(jnp.int32)
  src = ei[0]
  dst = ei[1]
  pad = E_PAD - E
  pad_idx = jnp.arange(pad, dtype=jnp.int32)
  src_p = jnp.concatenate([src, pad_idx % N])
  dst_p = jnp.concatenate([dst, N + pad_idx % (NT - N)])
  src3 = src_p.reshape(NS, NPASS, HALF, CH)
  dst3 = dst_p.reshape(NS, NPASS, HALF, CH)

  b1r = b1.reshape(1, D)
  b2r = b2.reshape(1, D)

  y1a, y1b, r1 = _tc_pre(x, W1_l, W1_r, b1r)
  h, cnt = _sc_agg_cnt(y1a, y1b, src3, dst3, r1)
  y2a, y2b, r2 = _tc_pre(h, W2_l, W2_r, b2r)
  return _sc_agg(y2a, y2b, src3, dst3, r2, cnt)
